# Initial kernel scaffold; baseline (speedup 1.0000x reference)
import jax
import jax.numpy as jnp
from jax.experimental import pallas as pl


def kernel(data, W, b, codebook):
    raise NotImplementedError("write your pallas kernel here")


# speedup vs baseline: 3.1088x; 3.1088x over previous
import functools

import jax
import jax.numpy as jnp
from jax.experimental import pallas as pl
from jax.experimental.pallas import tpu as pltpu


def _fused_kernel(x_ref, w_ref, b_ref, cb_ref, o_ref, csq_scr):
    i = pl.program_id(0)

    @pl.when(i == 0)
    def _prep():
        c = cb_ref[...]
        csq_scr[...] = jnp.sum(c * c, axis=1)[None, :]

    cf = jnp.dot(x_ref[...], w_ref[...],
                 preferred_element_type=jnp.float32) + b_ref[...]
    xsq = jnp.sum(cf * cf, axis=1, keepdims=True)
    cross = jax.lax.dot_general(
        cf, cb_ref[...], (((1,), (1,)), ((), ())),
        preferred_element_type=jnp.float32)
    pred = (xsq - 2.0 * cross) + csq_scr[...]
    logits = -pred
    m = jnp.max(logits, axis=1, keepdims=True)
    e = jnp.exp(logits - m)
    o_ref[...] = e * (1.0 / jnp.sum(e, axis=1, keepdims=True))


@jax.jit
def kernel(data, W, b, codebook):
    B, S, D_IN = data.shape
    NK, CODE_DIM = codebook.shape
    n_tok = B * S
    TB = 256
    x = data.reshape(n_tok, D_IN)

    out = pl.pallas_call(
        _fused_kernel,
        grid=(n_tok // TB,),
        in_specs=[
            pl.BlockSpec((TB, D_IN), lambda i: (i, 0)),
            pl.BlockSpec((D_IN, CODE_DIM), lambda i: (0, 0)),
            pl.BlockSpec((1, CODE_DIM), lambda i: (0, 0)),
            pl.BlockSpec((NK, CODE_DIM), lambda i: (0, 0)),
        ],
        out_specs=pl.BlockSpec((TB, NK), lambda i: (i, 0)),
        out_shape=jax.ShapeDtypeStruct((n_tok, NK), jnp.float32),
        scratch_shapes=[
            pltpu.VMEM((1, NK), jnp.float32),
        ],
        compiler_params=pltpu.CompilerParams(
            dimension_semantics=("arbitrary",),
        ),
    )(x, W, b.reshape(1, CODE_DIM), codebook)
    return out.reshape(B, S, NK)
